# Initial kernel scaffold; baseline (speedup 1.0000x reference)
import jax
import jax.numpy as jnp
from jax.experimental import pallas as pl


def kernel(x, W1, a_src1, a_dst1, b1, W2, a_src2, a_dst2, b2):
    raise NotImplementedError("write your pallas kernel here")


# speedup vs baseline: 3280.2334x; 3280.2334x over previous
import jax
import jax.numpy as jnp
from jax.experimental import pallas as pl

_N = 700
_NP = 704
_HEADS = 8
_HID = 8
_XD = 128


def _leaky_relu(v):
    return jnp.where(v >= 0, v, 0.2 * v)


def _attend(g, ad_vec, as_vec, src_valid):
    ad_col = ad_vec.reshape(_NP, 1)
    as_row = as_vec.reshape(1, _NP)
    e = _leaky_relu(ad_col + as_row)
    e = jnp.where(src_valid, e, -1e30)
    m = jnp.max(e, axis=1, keepdims=True)
    p = jnp.exp(e - m)
    denom = jnp.sum(p, axis=1, keepdims=True)
    alpha = p / (denom + 1e-16)
    return jnp.dot(alpha, g, preferred_element_type=jnp.float32)


def _gat_body(x_ref, w1_ref, as1_ref, ad1_ref, b1_ref,
              w2_ref, as2_ref, ad2_ref, b2_ref, o_ref):
    src_valid = jax.lax.broadcasted_iota(jnp.int32, (_NP, _NP), 1) < _N

    h = jnp.dot(x_ref[...], w1_ref[...], preferred_element_type=jnp.float32)
    outs = []
    for i in range(_HEADS):
        hi = h[:, i * _HID:(i + 1) * _HID]
        as_vec = jnp.sum(hi * as1_ref[i, :][None, :], axis=1)
        ad_vec = jnp.sum(hi * ad1_ref[i, :][None, :], axis=1)
        outs.append(_attend(hi, ad_vec, as_vec, src_valid))
    h1 = jnp.concatenate(outs, axis=1) + b1_ref[...]
    h1 = jnp.maximum(h1, 0.0)

    g = jnp.dot(h1, w2_ref[...], preferred_element_type=jnp.float32)
    as2 = jnp.sum(g * as2_ref[...], axis=1)
    ad2 = jnp.sum(g * ad2_ref[...], axis=1)
    o_ref[...] = _attend(g, ad2, as2, src_valid) + b2_ref[...]


def kernel(x, W1, a_src1, a_dst1, b1, W2, a_src2, a_dst2, b2):
    x_p = jnp.zeros((_NP, _XD), jnp.float32).at[:_N, :].set(x)
    out = pl.pallas_call(
        _gat_body,
        out_shape=jax.ShapeDtypeStruct((_NP, _XD), jnp.float32),
    )(x_p, W1, a_src1, a_dst1, b1.reshape(1, -1),
      W2, a_src2, a_dst2, b2.reshape(1, -1))
    return out[:_N, :]
